# Initial kernel scaffold; baseline (speedup 1.0000x reference)
import jax
import jax.numpy as jnp
from jax.experimental import pallas as pl


def kernel(x):
    raise NotImplementedError("write your pallas kernel here")


# speedup vs baseline: 2.6138x; 2.6138x over previous
import jax
import jax.numpy as jnp
from jax.experimental import pallas as pl

_ROWS, _COLS = 16384, 4096
_BLOCK_ROWS = 512
_STRIDE = 64


def _mask_copy_kernel(x_ref, o_ref):
    lane = jax.lax.broadcasted_iota(jnp.int32, (_BLOCK_ROWS, _COLS), 1)
    keep = (lane % _STRIDE) != 0
    o_ref[...] = jnp.where(keep, x_ref[...], 0.0)


def kernel(x):
    grid = (_ROWS // _BLOCK_ROWS,)
    return pl.pallas_call(
        _mask_copy_kernel,
        grid=grid,
        in_specs=[pl.BlockSpec((_BLOCK_ROWS, _COLS), lambda i: (i, 0))],
        out_specs=pl.BlockSpec((_BLOCK_ROWS, _COLS), lambda i: (i, 0)),
        out_shape=jax.ShapeDtypeStruct((_ROWS, _COLS), x.dtype),
    )(x)
